# baseline (speedup 1.0000x reference)
import jax
import jax.numpy as jnp
from jax.experimental import pallas as pl
from jax.experimental.pallas import tpu as pltpu

_CHUNKS = (4, 4, 4, 2, 4, 4, 4, 2, 4, 4, 4, 2, 4, 4, 4, 2, 4, 4)
_SLOTS = (4, 4, 4, 2)


def _copy_body(x_hbm, o_hbm, *args):
    nbuf = len(_SLOTS)
    bufs, (rsem, wsem) = args[:nbuf], args[nbuf:]
    nch = len(_CHUNKS)
    offs = [sum(_CHUNKS[:i]) for i in range(nch)]

    def read(i):
        b = i % nbuf
        return pltpu.make_async_copy(
            x_hbm.at[pl.ds(offs[i], _CHUNKS[i])],
            bufs[b].at[pl.ds(0, _CHUNKS[i])],
            rsem.at[b],
        )

    def write(i):
        b = i % nbuf
        return pltpu.make_async_copy(
            bufs[b].at[pl.ds(0, _CHUNKS[i])],
            o_hbm.at[pl.ds(offs[i], _CHUNKS[i])],
            wsem.at[b],
        )

    for i in range(nch):
        if i >= nbuf:
            write(i - nbuf).wait()
        read(i).start()
        if i >= 1:
            read(i - 1).wait()
            write(i - 1).start()
    read(nch - 1).wait()
    write(nch - 1).start()
    for j in range(nch - nbuf, nch):
        write(j).wait()


def kernel(x):
    b, n, f = x.shape
    return pl.pallas_call(
        _copy_body,
        out_shape=jax.ShapeDtypeStruct(x.shape, x.dtype),
        in_specs=[pl.BlockSpec(memory_space=pltpu.MemorySpace.HBM)],
        out_specs=pl.BlockSpec(memory_space=pltpu.MemorySpace.HBM),
        scratch_shapes=[pltpu.VMEM((s, n, f), x.dtype) for s in _SLOTS]
        + [
            pltpu.SemaphoreType.DMA((len(_SLOTS),)),
            pltpu.SemaphoreType.DMA((len(_SLOTS),)),
        ],
    )(x)


# speedup vs baseline: 1.0040x; 1.0040x over previous
import jax
import jax.numpy as jnp
from jax.experimental import pallas as pl
from jax.experimental.pallas import tpu as pltpu

_CHUNKS = (5, 5, 5, 5, 5, 5, 5, 5, 5, 5, 5, 5, 4)
_SLOTS = (5, 5, 5)


def _copy_body(x_hbm, o_hbm, *args):
    nbuf = len(_SLOTS)
    bufs, (rsem, wsem) = args[:nbuf], args[nbuf:]
    nch = len(_CHUNKS)
    offs = [sum(_CHUNKS[:i]) for i in range(nch)]

    def read(i):
        b = i % nbuf
        return pltpu.make_async_copy(
            x_hbm.at[pl.ds(offs[i], _CHUNKS[i])],
            bufs[b].at[pl.ds(0, _CHUNKS[i])],
            rsem.at[b],
        )

    def write(i):
        b = i % nbuf
        return pltpu.make_async_copy(
            bufs[b].at[pl.ds(0, _CHUNKS[i])],
            o_hbm.at[pl.ds(offs[i], _CHUNKS[i])],
            wsem.at[b],
        )

    for i in range(nch):
        if i >= nbuf:
            write(i - nbuf).wait()
        read(i).start()
        if i >= 1:
            read(i - 1).wait()
            write(i - 1).start()
    read(nch - 1).wait()
    write(nch - 1).start()
    for j in range(nch - nbuf, nch):
        write(j).wait()


def kernel(x):
    b, n, f = x.shape
    return pl.pallas_call(
        _copy_body,
        out_shape=jax.ShapeDtypeStruct(x.shape, x.dtype),
        in_specs=[pl.BlockSpec(memory_space=pltpu.MemorySpace.HBM)],
        out_specs=pl.BlockSpec(memory_space=pltpu.MemorySpace.HBM),
        compiler_params=pltpu.CompilerParams(vmem_limit_bytes=64 * 1024 * 1024),
        scratch_shapes=[pltpu.VMEM((s, n, f), x.dtype) for s in _SLOTS]
        + [
            pltpu.SemaphoreType.DMA((len(_SLOTS),)),
            pltpu.SemaphoreType.DMA((len(_SLOTS),)),
        ],
    )(x)
